# baseline (speedup 1.0000x reference)
import jax
import jax.numpy as jnp
import numpy as np
from jax import lax
from jax.experimental import pallas as pl
from jax.experimental.pallas import tpu as pltpu
from jax.experimental.pallas import tpu_sc as plsc

B = 4096
D_NUM = 100
D_PAD = 112
N_CAT = 26
CAT_SIZE = 1000
D_TOKEN = 128
N_TOK = D_NUM + N_CAT

NC = 2
NS = 16
NW = NC * NS
BW = B // NW
L = 16

_DSTN = (np.arange(B, dtype=np.int64)[:, None] * N_TOK
         + np.arange(D_NUM, dtype=np.int64)[None, :])
_DSTN3 = _DSTN.astype(np.int32).reshape(NW, BW, D_NUM)


def _sc_body(xcatT_hbm, xpad_hbm, dstn_hbm, w_hbm, table_hbm, out_hbm,
             w_v, x_v, dstn_v, idx_v, dstc_v, gbuf, tile, sem_g):
    w = lax.axis_index("c") * NS + lax.axis_index("s")

    pltpu.sync_copy(w_hbm, w_v)
    pltpu.sync_copy(xpad_hbm.at[w], x_v)
    pltpu.sync_copy(dstn_hbm.at[w], dstn_v)

    iota = lax.iota(jnp.int32, L)
    row0 = w * BW * N_TOK

    def cat_body(j, carry):
        pltpu.sync_copy(xcatT_hbm.at[j, w], idx_v)
        off = j * CAT_SIZE
        base = row0 + D_NUM + j
        for c in range(BW // L):
            s = pl.ds(c * L, L)
            idx_v[s] = idx_v[s] + off
            dstc_v[s] = iota * N_TOK + (base + c * L * N_TOK)
        pltpu.async_copy(table_hbm.at[idx_v], gbuf, sem_g).wait()
        pltpu.sync_copy(gbuf, out_hbm.at[dstc_v])
        return carry

    lax.fori_loop(0, N_CAT, cat_body, 0)

    def num_body(bl, carry):
        def emit_row(d):
            splat = plsc.load_gather(
                x_v, [jnp.full((L,), bl, jnp.int32), jnp.full((L,), d, jnp.int32)])
            for g in range(D_TOKEN // L):
                s = pl.ds(g * L, L)
                tile[d, s] = splat * w_v[d, s]

        def chunk_body(k, carry2):
            for dd in range(L):
                emit_row(k * L + dd)
            return carry2

        lax.fori_loop(0, D_NUM // L, chunk_body, 0)
        for dd in range(D_NUM % L):
            emit_row((D_NUM // L) * L + dd)

        pltpu.sync_copy(tile, out_hbm.at[dstn_v.at[bl]])
        return carry

    lax.fori_loop(0, BW, num_body, 0)


@jax.jit
def _sc_all(xcatT3, xpad3, dstn3, weight, cat_table):
    mesh = plsc.VectorSubcoreMesh(
        core_axis_name="c", subcore_axis_name="s", num_cores=NC, num_subcores=NS)
    return pl.kernel(
        _sc_body,
        out_type=jax.ShapeDtypeStruct((B * N_TOK, D_TOKEN), jnp.float32),
        mesh=mesh,
        compiler_params=pltpu.CompilerParams(needs_layout_passes=False),
        scratch_types=[
            pltpu.VMEM((D_NUM, D_TOKEN), jnp.float32),
            pltpu.VMEM((BW, D_PAD), jnp.float32),
            pltpu.VMEM((BW, D_NUM), jnp.int32),
            pltpu.VMEM((BW,), jnp.int32),
            pltpu.VMEM((BW,), jnp.int32),
            pltpu.VMEM((BW, D_TOKEN), jnp.float32),
            pltpu.VMEM((D_NUM, D_TOKEN), jnp.float32),
            pltpu.SemaphoreType.DMA,
        ],
    )(xcatT3, xpad3, dstn3, weight, cat_table)


def kernel(x_num, x_cat, weight, cat_table):
    xcatT3 = x_cat.T.reshape(N_CAT, NW, BW)
    xpad3 = jnp.pad(x_num, ((0, 0), (0, D_PAD - D_NUM))).reshape(NW, BW, D_PAD)
    dstn3 = jnp.asarray(_DSTN3)
    out = _sc_all(xcatT3, xpad3, dstn3, weight, cat_table)
    return out.reshape(B, N_TOK, D_TOKEN)


# speedup vs baseline: 1.0858x; 1.0858x over previous
import jax
import jax.numpy as jnp
import numpy as np
from jax import lax
from jax.experimental import pallas as pl
from jax.experimental.pallas import tpu as pltpu
from jax.experimental.pallas import tpu_sc as plsc

B = 4096
D_NUM = 100
N_CAT = 26
CAT_SIZE = 1000
D_TOKEN = 128
N_TOK = D_NUM + N_CAT

NC = 2
NS = 16
NW = NC * NS
BW = B // NW
L = 16
NCK = BW // L
NG = D_TOKEN // L

_LANE_IDS = [np.full(L, i, dtype=np.int32) for i in range(L)]


def _sc_body(xcatT_hbm, xT_hbm, w_hbm, table_hbm, out_hbm,
             w_v, x_v, idx0, idx1, dc0, dc1, base_v, dn0, dn1,
             gbuf0, gbuf1, tile0, tile1,
             sem_g0, sem_g1, sem_s0, sem_s1, sem_n0, sem_n1):
    w = lax.axis_index("c") * NS + lax.axis_index("s")

    pltpu.sync_copy(w_hbm, w_v)
    pltpu.sync_copy(xT_hbm.at[w], x_v)

    iota = lax.iota(jnp.int32, L)
    row0 = w * BW * N_TOK

    for c in range(NCK):
        base_v[pl.ds(c * L, L)] = iota * N_TOK + (row0 + c * L * N_TOK)

    idx_b = (idx0, idx1)
    dc_b = (dc0, dc1)
    gb_b = (gbuf0, gbuf1)
    sg_b = (sem_g0, sem_g1)
    ss_b = (sem_s0, sem_s1)

    def prep(j, b):
        pltpu.sync_copy(xcatT_hbm.at[j, w], idx_b[b])
        for c in range(NCK):
            s = pl.ds(c * L, L)
            idx_b[b][s] = idx_b[b][s] + j * CAT_SIZE
            dc_b[b][s] = base_v[s] + (D_NUM + j)

    gathers = [None, None]
    scatters = [None, None]
    prep(0, 0)
    gathers[0] = pltpu.async_copy(table_hbm.at[idx0], gbuf0, sem_g0)
    for j in range(N_CAT):
        b = j % 2
        nb = (j + 1) % 2
        if j + 1 < N_CAT:
            if scatters[nb] is not None:
                scatters[nb].wait()
            prep(j + 1, nb)
            gathers[nb] = pltpu.async_copy(table_hbm.at[idx_b[nb]], gb_b[nb], sg_b[nb])
        gathers[b].wait()
        scatters[b] = pltpu.async_copy(gb_b[b], out_hbm.at[dc_b[b]], ss_b[b])
    scatters[0].wait()
    scatters[1].wait()

    ti_b = (tile0, tile1)
    dn_b = (dn0, dn1)
    sn_b = (sem_n0, sem_n1)
    lane_ids = [jnp.full((L,), i, dtype=jnp.int32) for i in range(L)]

    def compute_tile(d, t):
        for c in range(NCK):
            s = pl.ds(c * L, L)
            dn_b[t][s] = base_v[s] + d
        wrow = [w_v[d, pl.ds(g * L, L)] for g in range(NG)]
        for c in range(NCK):
            xc = x_v[d, pl.ds(c * L, L)]
            for i in range(L):
                splat = jnp.take(xc, lane_ids[i])
                for g in range(NG):
                    ti_b[t][c * L + i, pl.ds(g * L, L)] = splat * wrow[g]

    def num_pair(p, carry):
        for t in range(2):
            d = p * 2 + t

            @pl.when(p > 0)
            def _drain():
                pltpu.make_async_copy(ti_b[t], out_hbm.at[dn_b[t]], sn_b[t]).wait()

            compute_tile(d, t)
            pltpu.async_copy(ti_b[t], out_hbm.at[dn_b[t]], sn_b[t])
        return carry

    lax.fori_loop(0, D_NUM // 2, num_pair, 0)
    pltpu.make_async_copy(tile0, out_hbm.at[dn0], sem_n0).wait()
    pltpu.make_async_copy(tile1, out_hbm.at[dn1], sem_n1).wait()


@jax.jit
def _sc_all(xcatT3, xT3, weight, cat_table):
    mesh = plsc.VectorSubcoreMesh(
        core_axis_name="c", subcore_axis_name="s", num_cores=NC, num_subcores=NS)
    return pl.kernel(
        _sc_body,
        out_type=jax.ShapeDtypeStruct((B * N_TOK, D_TOKEN), jnp.float32),
        mesh=mesh,
        compiler_params=pltpu.CompilerParams(needs_layout_passes=False),
        scratch_types=[
            pltpu.VMEM((D_NUM, D_TOKEN), jnp.float32),
            pltpu.VMEM((D_NUM, BW), jnp.float32),
            pltpu.VMEM((BW,), jnp.int32),
            pltpu.VMEM((BW,), jnp.int32),
            pltpu.VMEM((BW,), jnp.int32),
            pltpu.VMEM((BW,), jnp.int32),
            pltpu.VMEM((BW,), jnp.int32),
            pltpu.VMEM((BW,), jnp.int32),
            pltpu.VMEM((BW,), jnp.int32),
            pltpu.VMEM((BW, D_TOKEN), jnp.float32),
            pltpu.VMEM((BW, D_TOKEN), jnp.float32),
            pltpu.VMEM((BW, D_TOKEN), jnp.float32),
            pltpu.VMEM((BW, D_TOKEN), jnp.float32),
            pltpu.SemaphoreType.DMA,
            pltpu.SemaphoreType.DMA,
            pltpu.SemaphoreType.DMA,
            pltpu.SemaphoreType.DMA,
            pltpu.SemaphoreType.DMA,
            pltpu.SemaphoreType.DMA,
        ],
    )(xcatT3, xT3, weight, cat_table)


def kernel(x_num, x_cat, weight, cat_table):
    xcatT3 = x_cat.T.reshape(N_CAT, NW, BW)
    xT3 = x_num.T.reshape(D_NUM, NW, BW).transpose(1, 0, 2)
    out = _sc_all(xcatT3, xT3, weight, cat_table)
    return out.reshape(B, N_TOK, D_TOKEN)
